# baseline (speedup 1.0000x reference)
import jax
import jax.numpy as jnp
from jax import lax
from jax.experimental import pallas as pl
from jax.experimental.pallas import tpu as pltpu
from jax.experimental.pallas import tpu_sc as plsc

N_PTS = 524288
C_DIM = 96
N_CELLS = 262144
NG = 8
G_SHIFT = 15
GB = 8
NB = NG * GB
BUCKET = N_CELLS // NB
NC = 2
NS = 16
P = N_PTS // NS
GCAP = P + 16
MCAP = P + 128
CHUNK = 64
TS = BUCKET // NS
DC = 128
SB = 2048


def _body(feat_hbm, gi_hbm, out_hbm,
          gbuf, glist, mptr, lcnt, idbuf, gidxa, gidxb, lidx, rowbuf,
          dbuf, zc, ctmp2, inv, acc, scnt, sema):
    c = lax.axis_index("c")
    s = lax.axis_index("s")
    tbase = s * P
    iota16 = lax.iota(jnp.int32, 16)
    zeros16 = jnp.zeros((16,), jnp.float32)
    ones16 = jnp.ones((16,), jnp.float32)
    neg16 = jnp.full((16,), -1, jnp.int32)

    def zcrow(r, _):
        zc[r] = zeros16
        return 0
    lax.fori_loop(0, 16, zcrow, 0)

    def idrow(q, _):
        for i in range(8):
            idbuf[q, pl.ds(i * 16, 16)] = q * 128 + i * 16 + iota16
        return 0
    lax.fori_loop(0, (BUCKET // 16) // 128, idrow, 0)

    def group_body(gi, _):
        g = c * (NG // NC) + gi

        def stream_body(ch, gcnt):
            pltpu.sync_copy(gi_hbm.at[pl.ds(tbase + ch * SB, SB)], gbuf)

            def scan_body(i, cnt):
                v = gbuf[pl.ds(i * 16, 16)]
                m = jnp.right_shift(v, G_SHIFT) == g
                rel = ch * SB + i * 16 + iota16
                e = jnp.bitwise_or(
                    jnp.left_shift(jnp.bitwise_and(v, 32767), 15), rel)
                pos = cnt + plsc.cumsum(m.astype(jnp.int32)) - 1
                plsc.store_scatter(glist, [pos], e, mask=m)
                return cnt + jnp.sum(m.astype(jnp.int32))
            return lax.fori_loop(0, SB // 16, scan_body, gcnt)
        gcnt = lax.fori_loop(0, P // SB, stream_body, jnp.int32(0))
        plsc.store_scatter(glist, [gcnt + iota16], neg16)
        gsteps = jnp.right_shift(gcnt + 15, 4)

        def bucket_body(sub, _):
            b = g * GB + sub

            def zl(i, _):
                lcnt[i] = zeros16
                return 0
            lax.fori_loop(0, BUCKET // 16, zl, 0)

            def zrow(r, _):
                for q in range(C_DIM // 16):
                    dbuf[r, pl.ds(q * 16, 16)] = zeros16
                return 0
            lax.fori_loop(0, DC, zrow, 0)
            for kk in range(TS // DC):
                pltpu.sync_copy(dbuf, acc.at[pl.ds(s * TS + kk * DC, DC)])
            pltpu.sync_copy(zc, scnt.at[pl.ds(s * 16, 16)])
            plsc.subcore_barrier()

            def bscan(i, cnt):
                e = glist[pl.ds(i * 16, 16)]
                cell15 = jnp.right_shift(e, 15)
                m = jnp.logical_and(
                    jnp.right_shift(cell15, 12) == sub, e >= 0)
                loc = jnp.bitwise_and(cell15, BUCKET - 1)
                plsc.addupdate_scatter(
                    lcnt, [jnp.right_shift(loc, 4),
                           jnp.bitwise_and(loc, 15)], ones16, mask=m)
                packed = jnp.bitwise_or(
                    jnp.left_shift(loc, 15), jnp.bitwise_and(e, 32767))
                pos = cnt + plsc.cumsum(m.astype(jnp.int32)) - 1
                plsc.store_scatter(mptr, [pos], packed, mask=m)
                return cnt + jnp.sum(m.astype(jnp.int32))
            cnt = lax.fori_loop(0, gsteps, bscan, jnp.int32(0))

            def pad_body(q, _):
                plsc.store_scatter(mptr, [cnt + q * 16 + iota16], neg16)
                return 0
            lax.fori_loop(0, 8, pad_body, 0)

            ncnk2 = jnp.right_shift(cnt + (2 * CHUNK - 1), 7)

            def build(j, gidx_b, half):
                for q in range(CHUNK // 16):
                    r16 = mptr[pl.ds(j * CHUNK + q * 16, 16)]
                    valid = r16 >= 0
                    loc = jnp.where(valid,
                                    jnp.right_shift(r16, 15),
                                    jnp.int32(BUCKET))
                    rel = jnp.bitwise_and(r16, 32767)
                    gidx_b[pl.ds(q * 16, 16)] = tbase + rel
                    lidx[pl.ds(half * CHUNK + q * 16, 16)] = loc

            def flush_body(j2, _):
                build(2 * j2, gidxa, 0)
                cpa = pltpu.async_copy(
                    feat_hbm.at[gidxa], rowbuf.at[pl.ds(0, CHUNK)], sema)
                build(2 * j2 + 1, gidxb, 1)
                cpb = pltpu.async_copy(
                    feat_hbm.at[gidxb], rowbuf.at[pl.ds(CHUNK, CHUNK)],
                    sema)
                cpa.wait()
                cpb.wait()
                pltpu.sync_copy(rowbuf, acc.at[lidx], add=True)
                return 0
            lax.fori_loop(0, ncnk2, flush_body, 0)

            for q in range((BUCKET // 16) // 128):
                pltpu.sync_copy(lcnt.at[pl.ds(q * 128, 128)],
                                scnt.at[idbuf.at[q]], add=True)
            plsc.subcore_barrier()

            pltpu.sync_copy(scnt.at[pl.ds(s * 16, 16)], ctmp2)

            def invb(i, _):
                cv = ctmp2[i]
                inv[pl.ds(i * 16, 16)] = 1.0 / jnp.maximum(cv, 1.0)
                return 0
            lax.fori_loop(0, 16, invb, 0)

            def dchunk(kk, _):
                row0 = s * TS + kk * DC
                pltpu.sync_copy(acc.at[pl.ds(row0, DC)], dbuf)

                def drow(r, _):
                    ivv = plsc.load_gather(
                        inv, [jnp.full((16,), kk * DC + r, jnp.int32)])
                    for q in range(C_DIM // 16):
                        dbuf[r, pl.ds(q * 16, 16)] = (
                            dbuf[r, pl.ds(q * 16, 16)] * ivv)
                    return 0
                lax.fori_loop(0, DC, drow, 0)
                pltpu.sync_copy(dbuf,
                                out_hbm.at[pl.ds(b * BUCKET + row0, DC)])
                return 0
            lax.fori_loop(0, TS // DC, dchunk, 0)

            plsc.subcore_barrier()
            return 0
        lax.fori_loop(0, GB, bucket_body, 0)
        return 0
    lax.fori_loop(0, NG // NC, group_body, 0)


@jax.jit
def kernel(feat, grid_index):
    run = pl.kernel(
        _body,
        out_type=jax.ShapeDtypeStruct((N_CELLS, C_DIM), jnp.float32),
        mesh=plsc.VectorSubcoreMesh(core_axis_name="c", subcore_axis_name="s"),
        compiler_params=pltpu.CompilerParams(
            needs_layout_passes=False, use_tc_tiling_on_sc=False),
        scratch_types=[
            pltpu.VMEM((SB,), jnp.int32),
            pltpu.VMEM((GCAP,), jnp.int32),
            pltpu.VMEM((MCAP,), jnp.int32),
            pltpu.VMEM((BUCKET // 16, 16), jnp.float32),
            pltpu.VMEM(((BUCKET // 16) // 128, 128), jnp.int32),
            pltpu.VMEM((CHUNK,), jnp.int32),
            pltpu.VMEM((CHUNK,), jnp.int32),
            pltpu.VMEM((2 * CHUNK,), jnp.int32),
            pltpu.VMEM((2 * CHUNK, C_DIM), jnp.float32),
            pltpu.VMEM((DC, C_DIM), jnp.float32),
            pltpu.VMEM((16, 16), jnp.float32),
            pltpu.VMEM((16, 16), jnp.float32),
            pltpu.VMEM((TS,), jnp.float32),
            pltpu.VMEM_SHARED((BUCKET + 8, C_DIM), jnp.float32),
            pltpu.VMEM_SHARED((BUCKET // 16, 16), jnp.float32),
            pltpu.SemaphoreType.DMA,
        ],
    )
    return run(feat, grid_index)


# speedup vs baseline: 1.1293x; 1.1293x over previous
import jax
import jax.numpy as jnp
from jax import lax
from jax.experimental import pallas as pl
from jax.experimental.pallas import tpu as pltpu
from jax.experimental.pallas import tpu_sc as plsc

N_PTS = 524288
C_DIM = 96
N_CELLS = 262144
NG = 8
G_SHIFT = 15
GB = 4
NB = NG * GB
BUCKET = N_CELLS // NB
L_SHIFT = 13
NC = 2
NS = 16
P = N_PTS // NS
GCAP = P + 16
FB = 128
MCAP = 2 * FB
TS = BUCKET // NS
DC = 128
SB = 2048


def _body(feat_hbm, gi_hbm, out_hbm,
          gbuf, glist, mptr, lcnt, idbuf, gidxa, gidxb, lidx, rowbuf,
          dbuf, zc, ctmp2, inv, acc, scnt, sem_g, sem_s):
    c = lax.axis_index("c")
    s = lax.axis_index("s")
    tbase = s * P
    iota16 = lax.iota(jnp.int32, 16)
    zeros16 = jnp.zeros((16,), jnp.float32)
    ones16 = jnp.ones((16,), jnp.float32)
    neg16 = jnp.full((16,), -1, jnp.int32)

    def zcrow(r, _):
        zc[r] = zeros16
        return 0
    lax.fori_loop(0, 32, zcrow, 0)

    def idrow(q, _):
        for i in range(8):
            idbuf[q, pl.ds(i * 16, 16)] = q * 128 + i * 16 + iota16
        return 0
    lax.fori_loop(0, (BUCKET // 16) // 128, idrow, 0)

    def drain_scatter():
        pltpu.make_async_copy(rowbuf, acc.at[lidx], sem_s).wait()

    def flush_once(fcnt):
        @pl.when(fcnt > 0)
        def _():
            drain_scatter()
        for q in range(FB // 16):
            r16 = mptr[pl.ds(q * 16, 16)]
            valid = r16 >= 0
            loc = jnp.where(valid,
                            jnp.right_shift(r16, 15),
                            jnp.int32(BUCKET))
            rel = jnp.bitwise_and(r16, 32767)
            gi_half = gidxa if q < (FB // 32) else gidxb
            gi_half[pl.ds((q % (FB // 32)) * 16, 16)] = tbase + rel
            lidx[pl.ds(q * 16, 16)] = loc
        cpa = pltpu.async_copy(
            feat_hbm.at[gidxa], rowbuf.at[pl.ds(0, FB // 2)], sem_g)
        cpb = pltpu.async_copy(
            feat_hbm.at[gidxb], rowbuf.at[pl.ds(FB // 2, FB // 2)], sem_g)
        cpa.wait()
        cpb.wait()
        pltpu.async_copy(rowbuf, acc.at[lidx], sem_s, add=True)

    def group_body(gi, _):
        g = c * (NG // NC) + gi

        def stream_body(ch, gcnt):
            pltpu.sync_copy(gi_hbm.at[pl.ds(tbase + ch * SB, SB)], gbuf)

            def scan_body(i, cnt):
                v = gbuf[pl.ds(i * 16, 16)]
                m = jnp.right_shift(v, G_SHIFT) == g
                rel = ch * SB + i * 16 + iota16
                e = jnp.bitwise_or(
                    jnp.left_shift(jnp.bitwise_and(v, 32767), 15), rel)
                pos = cnt + plsc.cumsum(m.astype(jnp.int32)) - 1
                plsc.store_scatter(glist, [pos], e, mask=m)
                return cnt + jnp.sum(m.astype(jnp.int32))
            return lax.fori_loop(0, SB // 16, scan_body, gcnt)
        gcnt = lax.fori_loop(0, P // SB, stream_body, jnp.int32(0))
        plsc.store_scatter(glist, [gcnt + iota16], neg16)
        gsteps = jnp.right_shift(gcnt + 15, 4)

        def bucket_body(sub, _):
            b = g * GB + sub

            def zl(i, _):
                lcnt[i] = zeros16
                return 0
            lax.fori_loop(0, BUCKET // 16, zl, 0)

            def zrow(r, _):
                for q in range(C_DIM // 16):
                    dbuf[r, pl.ds(q * 16, 16)] = zeros16
                return 0
            lax.fori_loop(0, DC, zrow, 0)
            for kk in range(TS // DC):
                pltpu.sync_copy(dbuf, acc.at[pl.ds(s * TS + kk * DC, DC)])
            pltpu.sync_copy(zc, scnt.at[pl.ds(s * 32, 32)])
            plsc.subcore_barrier()

            def bscan(i, carry):
                cnt, fcnt = carry
                e = glist[pl.ds(i * 16, 16)]
                cell15 = jnp.right_shift(e, 15)
                m = jnp.logical_and(
                    jnp.right_shift(cell15, L_SHIFT) == sub, e >= 0)
                loc = jnp.bitwise_and(cell15, BUCKET - 1)
                plsc.addupdate_scatter(
                    lcnt, [jnp.right_shift(loc, 4),
                           jnp.bitwise_and(loc, 15)], ones16, mask=m)
                packed = jnp.bitwise_or(
                    jnp.left_shift(loc, 15), jnp.bitwise_and(e, 32767))
                pos = cnt + plsc.cumsum(m.astype(jnp.int32)) - 1
                plsc.store_scatter(mptr, [pos], packed, mask=m)
                cnt2 = cnt + jnp.sum(m.astype(jnp.int32))
                full = cnt2 >= FB

                @pl.when(full)
                def _():
                    flush_once(fcnt)
                    ov = mptr[pl.ds(FB, 16)]
                    mptr[pl.ds(0, 16)] = ov
                cnt3 = jnp.where(full, cnt2 - FB, cnt2)
                return cnt3, fcnt + full.astype(jnp.int32)
            cnt, fcnt = lax.fori_loop(0, gsteps, bscan,
                                      (jnp.int32(0), jnp.int32(0)))

            def pad_body(q, _):
                plsc.store_scatter(mptr, [cnt + q * 16 + iota16], neg16)
                return 0
            lax.fori_loop(0, 8, pad_body, 0)

            @pl.when(cnt > 0)
            def _():
                flush_once(fcnt)
            ffin = fcnt + (cnt > 0).astype(jnp.int32)

            @pl.when(ffin > 0)
            def _():
                drain_scatter()

            for q in range((BUCKET // 16) // 128):
                pltpu.sync_copy(lcnt.at[pl.ds(q * 128, 128)],
                                scnt.at[idbuf.at[q]], add=True)
            plsc.subcore_barrier()

            pltpu.sync_copy(scnt.at[pl.ds(s * 32, 32)], ctmp2)

            def invb(i, _):
                cv = ctmp2[i]
                inv[pl.ds(i * 16, 16)] = 1.0 / jnp.maximum(cv, 1.0)
                return 0
            lax.fori_loop(0, 32, invb, 0)

            def dchunk(kk, _):
                row0 = s * TS + kk * DC
                pltpu.sync_copy(acc.at[pl.ds(row0, DC)], dbuf)

                def drow(r, _):
                    ivv = plsc.load_gather(
                        inv, [jnp.full((16,), kk * DC + r, jnp.int32)])
                    for q in range(C_DIM // 16):
                        dbuf[r, pl.ds(q * 16, 16)] = (
                            dbuf[r, pl.ds(q * 16, 16)] * ivv)
                    return 0
                lax.fori_loop(0, DC, drow, 0)
                pltpu.sync_copy(dbuf,
                                out_hbm.at[pl.ds(b * BUCKET + row0, DC)])
                return 0
            lax.fori_loop(0, TS // DC, dchunk, 0)

            plsc.subcore_barrier()
            return 0
        lax.fori_loop(0, GB, bucket_body, 0)
        return 0
    lax.fori_loop(0, NG // NC, group_body, 0)


@jax.jit
def kernel(feat, grid_index):
    run = pl.kernel(
        _body,
        out_type=jax.ShapeDtypeStruct((N_CELLS, C_DIM), jnp.float32),
        mesh=plsc.VectorSubcoreMesh(core_axis_name="c", subcore_axis_name="s"),
        compiler_params=pltpu.CompilerParams(
            needs_layout_passes=False, use_tc_tiling_on_sc=False),
        scratch_types=[
            pltpu.VMEM((SB,), jnp.int32),
            pltpu.VMEM((GCAP,), jnp.int32),
            pltpu.VMEM((MCAP,), jnp.int32),
            pltpu.VMEM((BUCKET // 16, 16), jnp.float32),
            pltpu.VMEM(((BUCKET // 16) // 128, 128), jnp.int32),
            pltpu.VMEM((FB // 2,), jnp.int32),
            pltpu.VMEM((FB // 2,), jnp.int32),
            pltpu.VMEM((FB,), jnp.int32),
            pltpu.VMEM((FB, C_DIM), jnp.float32),
            pltpu.VMEM((DC, C_DIM), jnp.float32),
            pltpu.VMEM((32, 16), jnp.float32),
            pltpu.VMEM((32, 16), jnp.float32),
            pltpu.VMEM((TS,), jnp.float32),
            pltpu.VMEM_SHARED((BUCKET + 8, C_DIM), jnp.float32),
            pltpu.VMEM_SHARED((BUCKET // 16, 16), jnp.float32),
            pltpu.SemaphoreType.DMA,
            pltpu.SemaphoreType.DMA,
        ],
    )
    return run(feat, grid_index)
